# baseline (speedup 1.0000x reference)
import functools

import jax
import jax.numpy as jnp
from jax import lax
from jax.experimental import pallas as pl
from jax.experimental.pallas import tpu as pltpu
from jax.experimental.pallas import tpu_sc as plsc

NUM_ROWS = 1_000_000
DIM = 64
BATCH = 4096 * 200
NC, NS = 2, 16
NW = NC * NS
BPW = BATCH // NW
CHUNK = 128
NCH = BPW // CHUNK
NBUF = 4
NGRP = CHUNK // 16

_mesh = plsc.VectorSubcoreMesh(core_axis_name="c", subcore_axis_name="s")


def _wid():
    return lax.axis_index("s") * NC + lax.axis_index("c")


@functools.partial(
    pl.kernel,
    mesh=_mesh,
    out_type=jax.ShapeDtypeStruct((BATCH, DIM), jnp.float32),
    compiler_params=pltpu.CompilerParams(needs_layout_passes=False),
    scratch_types=[
        pltpu.VMEM((NCH, CHUNK), jnp.int32),
        pltpu.VMEM((NBUF, CHUNK), jnp.int32),
        pltpu.VMEM((CHUNK, 128), jnp.float32),
        pltpu.VMEM((CHUNK, 128), jnp.float32),
        pltpu.VMEM((CHUNK, 128), jnp.float32),
        pltpu.VMEM((CHUNK, 128), jnp.float32),
        pltpu.VMEM((CHUNK, DIM), jnp.float32),
        pltpu.VMEM((CHUNK, DIM), jnp.float32),
        pltpu.SemaphoreType.DMA,
        pltpu.SemaphoreType.DMA,
        pltpu.SemaphoreType.DMA,
        pltpu.SemaphoreType.DMA,
        pltpu.SemaphoreType.DMA,
        pltpu.SemaphoreType.DMA,
        pltpu.SemaphoreType.DMA,
    ],
)
def _gather(
    x_hbm, wp_hbm, out_hbm,
    idx_v, jbuf, r0, r1, r2, r3, ob0, ob1,
    semi, sg0, sg1, sg2, sg3, so0, so1,
):
    wid = _wid()
    rows = (r0, r1, r2, r3)
    sgs = (sg0, sg1, sg2, sg3)
    obs = (ob0, ob1)
    sos = (so0, so1)
    base = wid * BPW

    pltpu.make_async_copy(
        x_hbm.at[pl.ds(wid * NCH, NCH), :], idx_v, semi
    ).start()
    pltpu.make_async_copy(
        x_hbm.at[pl.ds(wid * NCH, NCH), :], idx_v, semi
    ).wait()

    lane = lax.iota(jnp.int32, 16)

    def shift_and_fire(j, p):
        for g in range(NGRP):
            jbuf[p, pl.ds(16 * g, 16)] = (
                idx_v[j, pl.ds(16 * g, 16)] >> 1
            )
        pltpu.make_async_copy(wp_hbm.at[jbuf.at[p]], rows[p], sgs[p]).start()

    for p in range(NBUF):
        shift_and_fire(p, p)

    def extract(j, p, q):
        def grp(g, carry):
            i0 = 16 * g
            idx16 = idx_v[j, pl.ds(i0, 16)]
            colbase = (idx16 & 1) << 6
            rowid = i0 + lane
            for cc in range(0, DIM, 16):
                for k in range(16):
                    v = plsc.load_gather(
                        rows[p], [rowid, colbase + (cc + k)]
                    )
                    plsc.store_scatter(
                        obs[q], [rowid, jnp.full((16,), cc + k, jnp.int32)], v
                    )
            return carry

        lax.fori_loop(0, NGRP, grp, 0)

    def body(i, carry):
        for p in range(NBUF):
            j = NBUF * i + p
            q = p % 2
            pltpu.make_async_copy(wp_hbm.at[jbuf.at[p]], rows[p], sgs[p]).wait()

            @pl.when(j >= 2)
            def _():
                pltpu.make_async_copy(
                    obs[q],
                    out_hbm.at[pl.ds(base + (j - 2) * CHUNK, CHUNK), :],
                    sos[q],
                ).wait()

            extract(j, p, q)
            pltpu.make_async_copy(
                obs[q],
                out_hbm.at[pl.ds(base + j * CHUNK, CHUNK), :],
                sos[q],
            ).start()

            @pl.when(j + NBUF < NCH)
            def _():
                shift_and_fire(j + NBUF, p)

        return carry

    lax.fori_loop(0, NCH // NBUF, body, 0)

    for q in range(2):
        j = NCH - 2 + q
        pltpu.make_async_copy(
            obs[q],
            out_hbm.at[pl.ds(base + j * CHUNK, CHUNK), :],
            sos[q],
        ).wait()


def kernel(x, weight):
    x2 = x.reshape(BATCH // CHUNK, CHUNK).astype(jnp.int32)
    wp = weight.reshape(NUM_ROWS // 2, 2 * DIM)
    out = _gather(x2, wp)
    return out.reshape(x.shape[0], x.shape[1], DIM)


# speedup vs baseline: 2.0617x; 2.0617x over previous
import functools

import jax
import jax.numpy as jnp
from jax import lax
from jax.experimental import pallas as pl
from jax.experimental.pallas import tpu as pltpu
from jax.experimental.pallas import tpu_sc as plsc

NUM_ROWS = 1_000_000
DIM = 64
NA, NT = 4096, 200
NC, NS = 2, 16
NW = NC * NS
APW = NA // NW
NBUF = 2

RCH = 200
NRCH = NUM_ROWS // RCH
G1 = 128
G2 = NT - G1

_mesh = plsc.VectorSubcoreMesh(core_axis_name="c", subcore_axis_name="s")


def _wid():
    return lax.axis_index("s") * NC + lax.axis_index("c")


@functools.partial(
    pl.kernel,
    mesh=_mesh,
    out_type=jax.ShapeDtypeStruct((NUM_ROWS, 2 * DIM), jnp.float32),
    compiler_params=pltpu.CompilerParams(needs_layout_passes=False),
    scratch_types=[
        pltpu.VMEM((RCH, DIM), jnp.float32),
        pltpu.VMEM((RCH, DIM), jnp.float32),
        pltpu.VMEM((RCH, 2 * DIM), jnp.float32),
        pltpu.VMEM((RCH, 2 * DIM), jnp.float32),
        pltpu.SemaphoreType.DMA,
        pltpu.SemaphoreType.DMA,
        pltpu.SemaphoreType.DMA,
        pltpu.SemaphoreType.DMA,
    ],
)
def _widen(w_hbm, wc_hbm, a0, a1, b0, b1, si0, si1, so0, so1):
    wid = _wid()
    bufa = (a0, a1)
    bufb = (b0, b1)
    sis = (si0, si1)
    sos = (so0, so1)

    def body(k, carry):
        for p in range(2):
            c = (2 * k + p) * NW + wid

            @pl.when(c < NRCH)
            def _():
                pltpu.make_async_copy(
                    w_hbm.at[pl.ds(c * RCH, RCH), :], bufa[p], sis[p]
                ).start()

        for p in range(2):
            c = (2 * k + p) * NW + wid
            cprev = c - 2 * NW

            @pl.when((cprev >= 0) & (cprev < NRCH))
            def _():
                pltpu.make_async_copy(
                    bufb[p], wc_hbm.at[pl.ds(0, RCH), :], sos[p]
                ).wait()

            @pl.when(c < NRCH)
            def _():
                pltpu.make_async_copy(
                    w_hbm.at[pl.ds(c * RCH, RCH), :], bufa[p], sis[p]
                ).wait()

                def repack(r, carry2):
                    for cc in range(0, DIM, 16):
                        bufb[p][r, pl.ds(cc, 16)] = bufa[p][r, pl.ds(cc, 16)]
                    return carry2

                lax.fori_loop(0, RCH, repack, 0)
                pltpu.make_async_copy(
                    bufb[p], wc_hbm.at[pl.ds(c * RCH, RCH), :], sos[p]
                ).start()

        return carry

    nk = (-(-NRCH // NW) + 1) // 2
    lax.fori_loop(0, nk, body, 0)

    for p in range(2):
        c = (2 * (nk - 1) + p) * NW + wid

        @pl.when(c < NRCH)
        def _():
            pltpu.make_async_copy(
                bufb[p], wc_hbm.at[pl.ds(0, RCH), :], sos[p]
            ).wait()


@functools.partial(
    pl.kernel,
    mesh=_mesh,
    out_type=jax.ShapeDtypeStruct((NA, NT, DIM), jnp.float32),
    compiler_params=pltpu.CompilerParams(needs_layout_passes=False),
    scratch_types=[
        pltpu.VMEM((NBUF, NT), jnp.int32),
        pltpu.VMEM((NT, 2 * DIM), jnp.float32),
        pltpu.VMEM((NT, 2 * DIM), jnp.float32),
        pltpu.VMEM((NT, DIM), jnp.float32),
        pltpu.VMEM((NT, DIM), jnp.float32),
        pltpu.SemaphoreType.DMA,
        pltpu.SemaphoreType.DMA,
        pltpu.SemaphoreType.DMA,
        pltpu.SemaphoreType.DMA,
        pltpu.SemaphoreType.DMA,
        pltpu.SemaphoreType.DMA,
    ],
)
def _gather(
    x_hbm, wc_hbm, out_hbm,
    jbuf, r0, r1, ob0, ob1,
    sj0, sj1, sg0, sg1, so0, so1,
):
    wid = _wid()
    rows = (r0, r1)
    obs = (ob0, ob1)
    sjs = (sj0, sj1)
    sgs = (sg0, sg1)
    sos = (so0, so1)
    abase = wid * APW

    def fire(j, p):
        pltpu.make_async_copy(x_hbm.at[abase + j], jbuf.at[p], sjs[p]).start()

    def start_gather(j, p):
        pltpu.make_async_copy(x_hbm.at[abase + j], jbuf.at[p], sjs[p]).wait()
        pltpu.make_async_copy(
            wc_hbm.at[jbuf.at[p, pl.ds(0, G1)]],
            rows[p].at[pl.ds(0, G1), :],
            sgs[p],
        ).start()
        pltpu.make_async_copy(
            wc_hbm.at[jbuf.at[p, pl.ds(G1, G2)]],
            rows[p].at[pl.ds(G1, G2), :],
            sgs[p],
        ).start()

    fire(0, 0)
    start_gather(0, 0)
    fire(1, 1)

    def body(i, carry):
        for p in range(NBUF):
            j = NBUF * i + p

            pltpu.make_async_copy(
                wc_hbm.at[jbuf.at[p, pl.ds(0, G1)]],
                rows[p].at[pl.ds(0, G1), :],
                sgs[p],
            ).wait()
            pltpu.make_async_copy(
                wc_hbm.at[jbuf.at[p, pl.ds(G1, G2)]],
                rows[p].at[pl.ds(G1, G2), :],
                sgs[p],
            ).wait()

            @pl.when(j + 1 < APW)
            def _():
                start_gather(j + 1, 1 - p)

            @pl.when(j + NBUF < APW)
            def _():
                fire(j + NBUF, p)

            @pl.when(j >= NBUF)
            def _():
                pltpu.make_async_copy(
                    obs[p], out_hbm.at[abase + j - NBUF], sos[p]
                ).wait()

            def compact(r, carry2):
                for cc in range(0, DIM, 16):
                    obs[p][r, pl.ds(cc, 16)] = rows[p][r, pl.ds(cc, 16)]
                return carry2

            lax.fori_loop(0, NT, compact, 0)
            pltpu.make_async_copy(
                obs[p], out_hbm.at[abase + j], sos[p]
            ).start()

        return carry

    lax.fori_loop(0, APW // NBUF, body, 0)

    for p in range(NBUF):
        pltpu.make_async_copy(
            obs[p], out_hbm.at[abase + APW - NBUF + p], sos[p]
        ).wait()


def kernel(x, weight):
    wc = _widen(weight)
    out = _gather(x.astype(jnp.int32), wc)
    return out
